# baseline (speedup 1.0000x reference)
import jax
import jax.numpy as jnp
from jax.experimental import pallas as pl
from jax.experimental.pallas import tpu as pltpu

_EPS = 1e-5
_BLK = 10000


def _nt(a, b):
    return jax.lax.dot_general(a, b, (((1,), (1,)), ((), ())),
                               preferred_element_type=jnp.float32)


def _tmm(a, b):
    return jax.lax.dot_general(a, b, (((0,), (0,)), ((), ())),
                               preferred_element_type=jnp.float32)


def _dotb(a, b):
    return jnp.dot(a.astype(jnp.bfloat16), b.astype(jnp.bfloat16),
                   preferred_element_type=jnp.float32)


def _stats1_body(xt_ref, et_ref, gx_ref, sx_ref, ge_ref, se_ref):
    i = pl.program_id(0)

    @pl.when(i == 0)
    def _():
        gx_ref[...] = jnp.zeros_like(gx_ref)
        sx_ref[...] = jnp.zeros_like(sx_ref)
        ge_ref[...] = jnp.zeros_like(ge_ref)
        se_ref[...] = jnp.zeros_like(se_ref)

    xt = xt_ref[0]
    et = et_ref[0]
    ones = jnp.ones((1, xt.shape[1]), jnp.float32)
    gx_ref[...] += _nt(xt, xt)
    sx_ref[...] += _nt(xt, ones)
    ge_ref[...] += _nt(et, et)
    se_ref[...] += _nt(et, ones)


def _fwd(xt_ref, et_ref, mux_ref, mue_ref, w1p_ref, w1e_ref, be1_ref,
         alpha_ref, w2_ref):
    xc = xt_ref[0] - mux_ref[...]
    ec = et_ref[0] - mue_ref[...]
    y = (_tmm(xc.astype(jnp.bfloat16), w1p_ref[...])
         + _tmm(ec.astype(jnp.bfloat16), w1e_ref[...])
         + be1_ref[...])
    p = jnp.maximum(y, y * alpha_ref[...])
    return _dotb(p, w2_ref[...])


def _stats2_body(xt_ref, et_ref, mux_ref, mue_ref, w1p_ref, w1e_ref,
                 be1_ref, alpha_ref, w2_ref, s2_ref, q2_ref):
    i = pl.program_id(0)

    @pl.when(i == 0)
    def _():
        s2_ref[...] = jnp.zeros_like(s2_ref)
        q2_ref[...] = jnp.zeros_like(q2_ref)

    z = _fwd(xt_ref, et_ref, mux_ref, mue_ref, w1p_ref, w1e_ref,
             be1_ref, alpha_ref, w2_ref)
    ones = jnp.ones((1, z.shape[0]), jnp.float32)
    s2_ref[...] += _dotb(ones, z)
    q2_ref[...] += _dotb(ones, z * z)


def _out_body(xt_ref, et_ref, mux_ref, mue_ref, w1p_ref, w1e_ref,
              be1_ref, alpha_ref, w2_ref, sh_ref, out_ref):
    z = _fwd(xt_ref, et_ref, mux_ref, mue_ref, w1p_ref, w1e_ref,
             be1_ref, alpha_ref, w2_ref)
    out_ref[...] = z + sh_ref[...]


def kernel(last, extra, W1p, b1p, g1p, be1p, a1p, W2p, b2p, g2p, be2p,
           W1e, b1e, g1e, be1e, a1e, W2e, b2e, g2e, be2e):
    n = last.shape[0]
    blk = _BLK
    nb = n // blk
    assert nb * blk == n
    inv_n = 1.0 / n

    xt = last.reshape(nb, blk, 3).swapaxes(1, 2)
    et = extra.reshape(nb, blk, 16).swapaxes(1, 2)

    def cspec(d):
        return pl.BlockSpec((1, d, blk), lambda i: (i, 0, 0))

    def fspec(shape):
        return pl.BlockSpec(shape, lambda i: (0, 0))

    params = dict(
        grid=(nb,),
        compiler_params=pltpu.CompilerParams(
            dimension_semantics=("arbitrary",)),
    )

    gx, sx, ge, se = pl.pallas_call(
        _stats1_body,
        in_specs=[cspec(3), cspec(16)],
        out_specs=[fspec((3, 3)), fspec((3, 1)),
                   fspec((16, 16)), fspec((16, 1))],
        out_shape=[jax.ShapeDtypeStruct((3, 3), jnp.float32),
                   jax.ShapeDtypeStruct((3, 1), jnp.float32),
                   jax.ShapeDtypeStruct((16, 16), jnp.float32),
                   jax.ShapeDtypeStruct((16, 1), jnp.float32)],
        **params,
    )(xt, et)

    def fold1(g, s, wT, gamma):
        mu = s * inv_n
        cov = g * inv_n - mu @ mu.T
        var = jnp.sum(wT * (cov @ wT), axis=0, keepdims=True)
        a = gamma.reshape(1, -1) * jax.lax.rsqrt(var + _EPS)
        return mu, wT * a

    mux, w1pf = fold1(gx, sx, W1p.T, g1p)
    mue, w1ef = fold1(ge, se, W1e.T, g1e)

    w1p_part = jnp.pad(w1pf, ((0, 0), (0, 64))).astype(jnp.bfloat16)
    w1e_part = jnp.pad(w1ef, ((0, 0), (192, 0))).astype(jnp.bfloat16)
    be1_row = jnp.concatenate([be1p, be1e]).reshape(1, -1)
    alpha_row = jnp.concatenate(
        [jnp.full((1, 192), a1p, jnp.float32),
         jnp.full((1, 64), a1e, jnp.float32)], axis=1)
    w2c = (jnp.pad(W2p.T, ((0, 64), (0, 32)))
           + jnp.pad(W2e.T, ((192, 0), (96, 0))))

    common_specs = [cspec(3), cspec(16), fspec((3, 1)), fspec((16, 1)),
                    fspec((3, 256)), fspec((16, 256)), fspec((1, 256)),
                    fspec((1, 256)), fspec((256, 128))]

    s2, q2 = pl.pallas_call(
        _stats2_body,
        in_specs=common_specs,
        out_specs=[fspec((1, 128)), fspec((1, 128))],
        out_shape=[jax.ShapeDtypeStruct((1, 128), jnp.float32),
                   jax.ShapeDtypeStruct((1, 128), jnp.float32)],
        **params,
    )(xt, et, mux, mue, w1p_part, w1e_part, be1_row, alpha_row,
      w2c.astype(jnp.bfloat16))

    m2r = s2 * inv_n
    v2 = q2 * inv_n - m2r * m2r
    g2row = jnp.concatenate([g2p, g2e]).reshape(1, -1)
    be2row = jnp.concatenate([be2p, be2e]).reshape(1, -1)
    sc2 = g2row * jax.lax.rsqrt(v2 + _EPS)
    sh2 = be2row - m2r * sc2
    w2out = (w2c * sc2).astype(jnp.bfloat16)

    return pl.pallas_call(
        _out_body,
        in_specs=common_specs + [fspec((1, 128))],
        out_specs=pl.BlockSpec((blk, 128), lambda i: (i, 0)),
        out_shape=jax.ShapeDtypeStruct((n, 128), jnp.float32),
        **params,
    )(xt, et, mux, mue, w1p_part, w1e_part, be1_row, alpha_row,
      w2out, sh2)


# speedup vs baseline: 1.1509x; 1.1509x over previous
import jax
import jax.numpy as jnp
from jax.experimental import pallas as pl
from jax.experimental.pallas import tpu as pltpu

_EPS = 1e-5
_BLK = 10000


def _nt(a, b):
    return jax.lax.dot_general(a, b, (((1,), (1,)), ((), ())),
                               preferred_element_type=jnp.float32)


def _tmm(a, b):
    return jax.lax.dot_general(a, b, (((0,), (0,)), ((), ())),
                               preferred_element_type=jnp.float32)


def _stats1_body(x_ref, g_ref):
    i = pl.program_id(0)

    @pl.when(i == 0)
    def _():
        g_ref[...] = jnp.zeros_like(g_ref)

    xt = x_ref[0]
    g_ref[...] += _nt(xt, xt)


def _stats2_body(x_ref, mu_ref, w1_ref, alpha_ref, w2_ref,
                 s2_ref, q2_ref, zc_ref):
    i = pl.program_id(0)

    @pl.when(i == 0)
    def _():
        s2_ref[...] = jnp.zeros_like(s2_ref)
        q2_ref[...] = jnp.zeros_like(q2_ref)

    xc = (x_ref[0] - mu_ref[...]).astype(jnp.bfloat16)
    y = _tmm(xc, w1_ref[...])
    yb = y.astype(jnp.bfloat16)
    p = jnp.maximum(yb, yb * alpha_ref[...])
    z = jnp.dot(p, w2_ref[...], preferred_element_type=jnp.float32)
    zb = z.astype(jnp.bfloat16)
    zc_ref[...] = zb
    ones = jnp.ones((1, zb.shape[0]), jnp.bfloat16)
    s2_ref[...] += jnp.dot(ones, zb, preferred_element_type=jnp.float32)
    q2_ref[...] += jnp.dot(ones, zb * zb,
                           preferred_element_type=jnp.float32)


def _out_body(zc_ref, sc_ref, sh_ref, out_ref):
    out_ref[...] = (zc_ref[...].astype(jnp.float32) * sc_ref[...]
                    + sh_ref[...])


def kernel(last, extra, W1p, b1p, g1p, be1p, a1p, W2p, b2p, g2p, be2p,
           W1e, b1e, g1e, be1e, a1e, W2e, b2e, g2e, be2e):
    n = last.shape[0]
    blk = _BLK
    nb = n // blk
    assert nb * blk == n
    inv_n = 1.0 / n

    xe = jnp.concatenate(
        [last, extra, jnp.ones((n, 1), jnp.float32)], axis=1)
    x20 = xe.reshape(nb, blk, 20).swapaxes(1, 2)

    def fspec(shape):
        return pl.BlockSpec(shape, lambda i: (0, 0))

    xspec = pl.BlockSpec((1, 20, blk), lambda i: (i, 0, 0))

    params = dict(
        grid=(nb,),
        compiler_params=pltpu.CompilerParams(
            dimension_semantics=("arbitrary",)),
    )

    g20 = pl.pallas_call(
        _stats1_body,
        in_specs=[xspec],
        out_specs=fspec((20, 20)),
        out_shape=jax.ShapeDtypeStruct((20, 20), jnp.float32),
        **params,
    )(x20)

    g = g20 * inv_n
    mu_full = g[:, 19:20]
    mu_c = jnp.concatenate([mu_full[:19], jnp.zeros((1, 1))], axis=0)

    def fold1(cov, wT, gamma):
        var = jnp.sum(wT * (cov @ wT), axis=0, keepdims=True)
        a = gamma.reshape(1, -1) * jax.lax.rsqrt(var + _EPS)
        return wT * a

    mux = mu_full[0:3]
    mue = mu_full[3:19]
    covx = g[0:3, 0:3] - mux @ mux.T
    cove = g[3:19, 3:19] - mue @ mue.T
    w1pf = fold1(covx, W1p.T, g1p)
    w1ef = fold1(cove, W1e.T, g1e)
    be1_row = jnp.concatenate([be1p, be1e]).reshape(1, -1)

    w1_comb = (jnp.zeros((20, 256), jnp.float32)
               .at[0:3, 0:192].set(w1pf)
               .at[3:19, 192:256].set(w1ef)
               .at[19:20, :].set(be1_row)).astype(jnp.bfloat16)
    alpha_row = jnp.concatenate(
        [jnp.full((1, 192), a1p, jnp.float32),
         jnp.full((1, 64), a1e, jnp.float32)],
        axis=1).astype(jnp.bfloat16)
    w2c = (jnp.pad(W2p.T, ((0, 64), (0, 32)))
           + jnp.pad(W2e.T, ((192, 0), (96, 0))))

    s2, q2, zc = pl.pallas_call(
        _stats2_body,
        in_specs=[xspec, fspec((20, 1)), fspec((20, 256)),
                  fspec((1, 256)), fspec((256, 128))],
        out_specs=[fspec((1, 128)), fspec((1, 128)),
                   pl.BlockSpec((blk, 128), lambda i: (i, 0))],
        out_shape=[jax.ShapeDtypeStruct((1, 128), jnp.float32),
                   jax.ShapeDtypeStruct((1, 128), jnp.float32),
                   jax.ShapeDtypeStruct((n, 128), jnp.bfloat16)],
        **params,
    )(x20, mu_c, w1_comb, alpha_row, w2c.astype(jnp.bfloat16))

    m2r = s2 * inv_n
    v2 = q2 * inv_n - m2r * m2r
    g2row = jnp.concatenate([g2p, g2e]).reshape(1, -1)
    be2row = jnp.concatenate([be2p, be2e]).reshape(1, -1)
    sc2 = g2row * jax.lax.rsqrt(v2 + _EPS)
    sh2 = be2row - m2r * sc2

    return pl.pallas_call(
        _out_body,
        in_specs=[pl.BlockSpec((blk, 128), lambda i: (i, 0)),
                  fspec((1, 128)), fspec((1, 128))],
        out_specs=pl.BlockSpec((blk, 128), lambda i: (i, 0)),
        out_shape=jax.ShapeDtypeStruct((n, 128), jnp.float32),
        **params,
    )(zc, sc2, sh2)


# speedup vs baseline: 1.3495x; 1.1726x over previous
import jax
import jax.numpy as jnp
from jax.experimental import pallas as pl
from jax.experimental.pallas import tpu as pltpu

_EPS = 1e-5
_BLK = 10000


def _nt(a, b):
    return jax.lax.dot_general(a, b, (((1,), (1,)), ((), ())),
                               preferred_element_type=jnp.float32)


def _tmm(a, b):
    return jax.lax.dot_general(a, b, (((0,), (0,)), ((), ())),
                               preferred_element_type=jnp.float32)


def _stats1_body(x_ref, g_ref):
    i = pl.program_id(0)

    @pl.when(i == 0)
    def _():
        g_ref[...] = jnp.zeros_like(g_ref)

    xt = x_ref[0]
    g_ref[...] += _nt(xt, xt)


def _stats2_body(x_ref, mu_ref, w1_ref, alpha_ref, w2_ref,
                 s2_ref, q2_ref, zc_ref):
    i = pl.program_id(0)

    @pl.when(i == 0)
    def _():
        s2_ref[...] = jnp.zeros_like(s2_ref)
        q2_ref[...] = jnp.zeros_like(q2_ref)

    xc = (x_ref[0] - mu_ref[...]).astype(jnp.bfloat16)
    y = _tmm(xc, w1_ref[...])
    yb = y.astype(jnp.bfloat16)
    p = jnp.maximum(yb, yb * alpha_ref[...])
    z = jnp.dot(p, w2_ref[...], preferred_element_type=jnp.float32)
    zb = z.astype(jnp.bfloat16)
    zc_ref[...] = zb
    ones = jnp.ones((1, zb.shape[0]), jnp.bfloat16)
    s2_ref[...] += jnp.dot(ones, zb, preferred_element_type=jnp.float32)
    q2_ref[...] += jnp.dot(ones, zb * zb,
                           preferred_element_type=jnp.float32)


def _out_body(zc_ref, sc_ref, sh_ref, out_ref):
    out_ref[...] = (zc_ref[...].astype(jnp.float32) * sc_ref[...]
                    + sh_ref[...])


def kernel(last, extra, W1p, b1p, g1p, be1p, a1p, W2p, b2p, g2p, be2p,
           W1e, b1e, g1e, be1e, a1e, W2e, b2e, g2e, be2e):
    n = last.shape[0]
    blk = _BLK
    nb = n // blk
    assert nb * blk == n
    inv_n = 1.0 / n

    xp = last.reshape(nb, blk, 3).swapaxes(1, 2)
    ep = extra.reshape(nb, blk, 16).swapaxes(1, 2)
    x20 = jnp.concatenate(
        [xp, ep, jnp.ones((nb, 1, blk), jnp.float32)], axis=1)

    def fspec(shape):
        return pl.BlockSpec(shape, lambda i: (0, 0))

    xspec = pl.BlockSpec((1, 20, blk), lambda i: (i, 0, 0))

    params = dict(
        grid=(nb,),
        compiler_params=pltpu.CompilerParams(
            dimension_semantics=("arbitrary",)),
    )

    g20 = pl.pallas_call(
        _stats1_body,
        in_specs=[xspec],
        out_specs=fspec((20, 20)),
        out_shape=jax.ShapeDtypeStruct((20, 20), jnp.float32),
        **params,
    )(x20)

    g = g20 * inv_n
    mu_full = g[:, 19:20]
    mu_c = jnp.concatenate([mu_full[:19], jnp.zeros((1, 1))], axis=0)

    def fold1(cov, wT, gamma):
        var = jnp.sum(wT * (cov @ wT), axis=0, keepdims=True)
        a = gamma.reshape(1, -1) * jax.lax.rsqrt(var + _EPS)
        return wT * a

    mux = mu_full[0:3]
    mue = mu_full[3:19]
    covx = g[0:3, 0:3] - mux @ mux.T
    cove = g[3:19, 3:19] - mue @ mue.T
    w1pf = fold1(covx, W1p.T, g1p)
    w1ef = fold1(cove, W1e.T, g1e)
    be1_row = jnp.concatenate([be1p, be1e]).reshape(1, -1)

    w1_comb = (jnp.zeros((20, 256), jnp.float32)
               .at[0:3, 0:192].set(w1pf)
               .at[3:19, 192:256].set(w1ef)
               .at[19:20, :].set(be1_row)).astype(jnp.bfloat16)
    alpha_row = jnp.concatenate(
        [jnp.full((1, 192), a1p, jnp.float32),
         jnp.full((1, 64), a1e, jnp.float32)],
        axis=1).astype(jnp.bfloat16)
    w2c = (jnp.pad(W2p.T, ((0, 64), (0, 32)))
           + jnp.pad(W2e.T, ((192, 0), (96, 0))))

    s2, q2, zc = pl.pallas_call(
        _stats2_body,
        in_specs=[xspec, fspec((20, 1)), fspec((20, 256)),
                  fspec((1, 256)), fspec((256, 128))],
        out_specs=[fspec((1, 128)), fspec((1, 128)),
                   pl.BlockSpec((blk, 128), lambda i: (i, 0))],
        out_shape=[jax.ShapeDtypeStruct((1, 128), jnp.float32),
                   jax.ShapeDtypeStruct((1, 128), jnp.float32),
                   jax.ShapeDtypeStruct((n, 128), jnp.bfloat16)],
        **params,
    )(x20, mu_c, w1_comb, alpha_row, w2c.astype(jnp.bfloat16))

    m2r = s2 * inv_n
    v2 = q2 * inv_n - m2r * m2r
    g2row = jnp.concatenate([g2p, g2e]).reshape(1, -1)
    be2row = jnp.concatenate([be2p, be2e]).reshape(1, -1)
    sc2 = g2row * jax.lax.rsqrt(v2 + _EPS)
    sh2 = be2row - m2r * sc2

    return pl.pallas_call(
        _out_body,
        in_specs=[pl.BlockSpec((blk, 128), lambda i: (i, 0)),
                  fspec((1, 128)), fspec((1, 128))],
        out_specs=pl.BlockSpec((blk, 128), lambda i: (i, 0)),
        out_shape=jax.ShapeDtypeStruct((n, 128), jnp.float32),
        **params,
    )(zc, sc2, sh2)


# speedup vs baseline: 1.3645x; 1.0111x over previous
import jax
import jax.numpy as jnp
from jax.experimental import pallas as pl
from jax.experimental.pallas import tpu as pltpu

_EPS = 1e-5
_BLK = 10000


def _nt(a, b):
    return jax.lax.dot_general(a, b, (((1,), (1,)), ((), ())),
                               preferred_element_type=jnp.float32)


def _tmm(a, b):
    return jax.lax.dot_general(a, b, (((0,), (0,)), ((), ())),
                               preferred_element_type=jnp.float32)


def _stats1_body(x_ref, g_ref):
    i = pl.program_id(0)

    @pl.when(i == 0)
    def _():
        g_ref[...] = jnp.zeros_like(g_ref)

    xt = x_ref[0]
    g_ref[...] += _nt(xt, xt)


def _stats2_body(x_ref, mu_ref, w1_ref, alpha_ref, w2_ref,
                 s2_ref, q2_ref, zc_ref):
    i = pl.program_id(0)

    @pl.when(i == 0)
    def _():
        s2_ref[...] = jnp.zeros_like(s2_ref)
        q2_ref[...] = jnp.zeros_like(q2_ref)

    xc = (x_ref[0] - mu_ref[...]).astype(jnp.bfloat16)
    yb = _tmm(xc, w1_ref[...]).astype(jnp.bfloat16)
    p = jnp.maximum(yb, yb * alpha_ref[...])
    zb = jnp.dot(p, w2_ref[...],
                 preferred_element_type=jnp.float32).astype(jnp.bfloat16)
    zc_ref[...] = zb
    ones = jnp.ones((1, zb.shape[0]), jnp.bfloat16)
    s2_ref[...] += jnp.dot(ones, zb, preferred_element_type=jnp.float32)
    q2_ref[...] += jnp.dot(ones, zb * zb,
                           preferred_element_type=jnp.float32)


def _out_body(zc_ref, s2_ref, q2_ref, g2_ref, be2_ref, out_ref, *, inv_n):
    m = s2_ref[...] * inv_n
    v = q2_ref[...] * inv_n - m * m
    sc = g2_ref[...] * jax.lax.rsqrt(v + _EPS)
    sh = be2_ref[...] - m * sc
    out_ref[...] = zc_ref[...].astype(jnp.float32) * sc + sh


def kernel(last, extra, W1p, b1p, g1p, be1p, a1p, W2p, b2p, g2p, be2p,
           W1e, b1e, g1e, be1e, a1e, W2e, b2e, g2e, be2e):
    n = last.shape[0]
    blk = _BLK
    nb = n // blk
    assert nb * blk == n
    inv_n = 1.0 / n

    xp = last.reshape(nb, blk, 3).swapaxes(1, 2)
    ep = extra.reshape(nb, blk, 16).swapaxes(1, 2)
    x20 = jnp.concatenate(
        [xp, ep, jnp.ones((nb, 1, blk), jnp.float32)], axis=1)

    def fspec(shape):
        return pl.BlockSpec(shape, lambda i: (0, 0))

    xspec = pl.BlockSpec((1, 20, blk), lambda i: (i, 0, 0))

    params = dict(
        grid=(nb,),
        compiler_params=pltpu.CompilerParams(
            dimension_semantics=("arbitrary",)),
    )

    g20 = pl.pallas_call(
        _stats1_body,
        in_specs=[xspec],
        out_specs=fspec((20, 20)),
        out_shape=jax.ShapeDtypeStruct((20, 20), jnp.float32),
        **params,
    )(x20)

    g = g20 * inv_n
    mu_full = g[:, 19:20]
    mu_c = jnp.concatenate([mu_full[:19], jnp.zeros((1, 1))], axis=0)

    def fold1(cov, wT, gamma):
        var = jnp.sum(wT * (cov @ wT), axis=0, keepdims=True)
        a = gamma.reshape(1, -1) * jax.lax.rsqrt(var + _EPS)
        return wT * a

    mux = mu_full[0:3]
    mue = mu_full[3:19]
    covx = g[0:3, 0:3] - mux @ mux.T
    cove = g[3:19, 3:19] - mue @ mue.T
    w1pf = fold1(covx, W1p.T, g1p)
    w1ef = fold1(cove, W1e.T, g1e)
    be1_row = jnp.concatenate([be1p, be1e]).reshape(1, -1)

    w1_comb = (jnp.zeros((20, 256), jnp.float32)
               .at[0:3, 0:192].set(w1pf)
               .at[3:19, 192:256].set(w1ef)
               .at[19:20, :].set(be1_row)).astype(jnp.bfloat16)
    alpha_row = jnp.concatenate(
        [jnp.full((1, 192), a1p, jnp.float32),
         jnp.full((1, 64), a1e, jnp.float32)],
        axis=1).astype(jnp.bfloat16)
    w2c = (jnp.pad(W2p.T, ((0, 64), (0, 32)))
           + jnp.pad(W2e.T, ((192, 0), (96, 0))))

    s2, q2, zc = pl.pallas_call(
        _stats2_body,
        in_specs=[xspec, fspec((20, 1)), fspec((20, 256)),
                  fspec((1, 256)), fspec((256, 128))],
        out_specs=[fspec((1, 128)), fspec((1, 128)),
                   pl.BlockSpec((blk, 128), lambda i: (i, 0))],
        out_shape=[jax.ShapeDtypeStruct((1, 128), jnp.float32),
                   jax.ShapeDtypeStruct((1, 128), jnp.float32),
                   jax.ShapeDtypeStruct((n, 128), jnp.bfloat16)],
        **params,
    )(x20, mu_c, w1_comb, alpha_row, w2c.astype(jnp.bfloat16))

    g2row = jnp.concatenate([g2p, g2e]).reshape(1, -1)
    be2row = jnp.concatenate([be2p, be2e]).reshape(1, -1)
    import functools as _ft
    return pl.pallas_call(
        _ft.partial(_out_body, inv_n=inv_n),
        in_specs=[pl.BlockSpec((blk, 128), lambda i: (i, 0)),
                  fspec((1, 128)), fspec((1, 128)),
                  fspec((1, 128)), fspec((1, 128))],
        out_specs=pl.BlockSpec((blk, 128), lambda i: (i, 0)),
        out_shape=jax.ShapeDtypeStruct((n, 128), jnp.float32),
        **params,
    )(zc, s2, q2, g2row, be2row)


# speedup vs baseline: 1.3929x; 1.0208x over previous
import jax
import jax.numpy as jnp
from jax.experimental import pallas as pl
from jax.experimental.pallas import tpu as pltpu

_EPS = 1e-5
_BLK = 10000


def _nt(a, b):
    return jax.lax.dot_general(a, b, (((1,), (1,)), ((), ())),
                               preferred_element_type=jnp.float32)


def _tmm(a, b):
    return jax.lax.dot_general(a, b, (((0,), (0,)), ((), ())),
                               preferred_element_type=jnp.float32)


def _stats1_body(x_ref, g_ref):
    i = pl.program_id(0)

    @pl.when(i == 0)
    def _():
        g_ref[...] = jnp.zeros_like(g_ref)

    xt = x_ref[0]
    g_ref[...] += _nt(xt, xt)


def _stats2_body(x_ref, g_ref, w1p_ref, w1e_ref, g1p_ref, g1e_ref,
                 be1_ref, alpha_ref, w2_ref, s2_ref, q2_ref, zc_ref,
                 mu_ref, w1_ref, *, inv_n):
    i = pl.program_id(0)

    @pl.when(i == 0)
    def _():
        s2_ref[...] = jnp.zeros_like(s2_ref)
        q2_ref[...] = jnp.zeros_like(q2_ref)
        g = g_ref[...] * inv_n
        mu_full = g[:, 19:20]
        keep = jax.lax.broadcasted_iota(jnp.int32, (20, 1), 0) < 19
        mu_ref[...] = jnp.where(keep, mu_full, 0.0)

        def scaled(cov, wT, gamma):
            var = jnp.sum(wT * (cov @ wT), axis=0, keepdims=True)
            a = gamma * jax.lax.rsqrt(var + _EPS)
            return (wT * a).astype(jnp.bfloat16)

        mux = mu_full[0:3]
        mue = mu_full[3:19]
        w1_ref[...] = jnp.zeros_like(w1_ref)
        w1_ref[0:3, 0:192] = scaled(g[0:3, 0:3] - mux @ mux.T,
                                    w1p_ref[...], g1p_ref[...])
        w1_ref[3:19, 192:256] = scaled(g[3:19, 3:19] - mue @ mue.T,
                                       w1e_ref[...], g1e_ref[...])
        w1_ref[19:20, :] = be1_ref[...].astype(jnp.bfloat16)

    xc = (x_ref[0] - mu_ref[...]).astype(jnp.bfloat16)
    yb = _tmm(xc, w1_ref[...]).astype(jnp.bfloat16)
    p = jnp.maximum(yb, yb * alpha_ref[...])
    zb = jnp.dot(p, w2_ref[...],
                 preferred_element_type=jnp.float32).astype(jnp.bfloat16)
    zc_ref[...] = zb
    ones = jnp.ones((1, zb.shape[0]), jnp.bfloat16)
    s2_ref[...] += jnp.dot(ones, zb, preferred_element_type=jnp.float32)
    q2_ref[...] += jnp.dot(ones, zb * zb,
                           preferred_element_type=jnp.float32)


def _out_body(zc_ref, s2_ref, q2_ref, g2_ref, be2_ref, out_ref, *, inv_n):
    m = s2_ref[...] * inv_n
    v = q2_ref[...] * inv_n - m * m
    sc = g2_ref[...] * jax.lax.rsqrt(v + _EPS)
    sh = be2_ref[...] - m * sc
    out_ref[...] = zc_ref[...].astype(jnp.float32) * sc + sh


def kernel(last, extra, W1p, b1p, g1p, be1p, a1p, W2p, b2p, g2p, be2p,
           W1e, b1e, g1e, be1e, a1e, W2e, b2e, g2e, be2e):
    n = last.shape[0]
    blk = _BLK
    nb = n // blk
    assert nb * blk == n
    inv_n = 1.0 / n

    xp = last.reshape(nb, blk, 3).swapaxes(1, 2)
    ep = extra.reshape(nb, blk, 16).swapaxes(1, 2)
    x20 = jnp.concatenate(
        [xp, ep, jnp.ones((nb, 1, blk), jnp.float32)], axis=1)

    def fspec(shape):
        return pl.BlockSpec(shape, lambda i: (0, 0))

    xspec = pl.BlockSpec((1, 20, blk), lambda i: (i, 0, 0))

    params = dict(
        grid=(nb,),
        compiler_params=pltpu.CompilerParams(
            dimension_semantics=("arbitrary",)),
    )

    g20 = pl.pallas_call(
        _stats1_body,
        in_specs=[xspec],
        out_specs=fspec((20, 20)),
        out_shape=jax.ShapeDtypeStruct((20, 20), jnp.float32),
        **params,
    )(x20)

    be1_row = jnp.concatenate([be1p, be1e]).reshape(1, -1)
    alpha_row = jnp.concatenate(
        [jnp.full((1, 192), a1p, jnp.float32),
         jnp.full((1, 64), a1e, jnp.float32)],
        axis=1).astype(jnp.bfloat16)
    w2c = (jnp.pad(W2p.T, ((0, 64), (0, 32)))
           + jnp.pad(W2e.T, ((192, 0), (96, 0))))

    import functools as _ft
    s2, q2, zc = pl.pallas_call(
        _ft.partial(_stats2_body, inv_n=inv_n),
        in_specs=[xspec, fspec((20, 20)), fspec((3, 192)), fspec((16, 64)),
                  fspec((1, 192)), fspec((1, 64)), fspec((1, 256)),
                  fspec((1, 256)), fspec((256, 128))],
        out_specs=[fspec((1, 128)), fspec((1, 128)),
                   pl.BlockSpec((blk, 128), lambda i: (i, 0))],
        out_shape=[jax.ShapeDtypeStruct((1, 128), jnp.float32),
                   jax.ShapeDtypeStruct((1, 128), jnp.float32),
                   jax.ShapeDtypeStruct((n, 128), jnp.bfloat16)],
        scratch_shapes=[pltpu.VMEM((20, 1), jnp.float32),
                        pltpu.VMEM((20, 256), jnp.bfloat16)],
        **params,
    )(x20, g20, W1p.T, W1e.T, g1p.reshape(1, -1), g1e.reshape(1, -1),
      be1_row, alpha_row, w2c.astype(jnp.bfloat16))

    g2row = jnp.concatenate([g2p, g2e]).reshape(1, -1)
    be2row = jnp.concatenate([be2p, be2e]).reshape(1, -1)
    import functools as _ft
    return pl.pallas_call(
        _ft.partial(_out_body, inv_n=inv_n),
        in_specs=[pl.BlockSpec((blk, 128), lambda i: (i, 0)),
                  fspec((1, 128)), fspec((1, 128)),
                  fspec((1, 128)), fspec((1, 128))],
        out_specs=pl.BlockSpec((blk, 128), lambda i: (i, 0)),
        out_shape=jax.ShapeDtypeStruct((n, 128), jnp.float32),
        **params,
    )(zc, s2, q2, g2row, be2row)


# speedup vs baseline: 1.4625x; 1.0500x over previous
import jax
import jax.numpy as jnp
from jax.experimental import pallas as pl
from jax.experimental.pallas import tpu as pltpu

_EPS = 1e-5
_BLK = 20000


def _nt(a, b):
    return jax.lax.dot_general(a, b, (((1,), (1,)), ((), ())),
                               preferred_element_type=jnp.float32)


def _tmm(a, b):
    return jax.lax.dot_general(a, b, (((0,), (0,)), ((), ())),
                               preferred_element_type=jnp.float32)


def _stats1_body(x_ref, g_ref):
    i = pl.program_id(0)

    @pl.when(i == 0)
    def _():
        g_ref[...] = jnp.zeros_like(g_ref)

    xt = x_ref[0]
    g_ref[...] += _nt(xt, xt)


def _stats2_body(x_ref, g_ref, w1p_ref, w1e_ref, g1p_ref, g1e_ref,
                 be1_ref, alpha_ref, w2_ref, s2_ref, q2_ref, zc_ref,
                 mu_ref, w1_ref, *, inv_n):
    i = pl.program_id(0)

    @pl.when(i == 0)
    def _():
        s2_ref[...] = jnp.zeros_like(s2_ref)
        q2_ref[...] = jnp.zeros_like(q2_ref)
        g = g_ref[...] * inv_n
        mu_full = g[:, 19:20]
        keep = jax.lax.broadcasted_iota(jnp.int32, (20, 1), 0) < 19
        mu_ref[...] = jnp.where(keep, mu_full, 0.0)

        def scaled(cov, wT, gamma):
            var = jnp.sum(wT * (cov @ wT), axis=0, keepdims=True)
            a = gamma * jax.lax.rsqrt(var + _EPS)
            return (wT * a).astype(jnp.bfloat16)

        mux = mu_full[0:3]
        mue = mu_full[3:19]
        w1_ref[...] = jnp.zeros_like(w1_ref)
        w1_ref[0:3, 0:192] = scaled(g[0:3, 0:3] - mux @ mux.T,
                                    w1p_ref[...], g1p_ref[...])
        w1_ref[3:19, 192:256] = scaled(g[3:19, 3:19] - mue @ mue.T,
                                       w1e_ref[...], g1e_ref[...])
        w1_ref[19:20, :] = be1_ref[...].astype(jnp.bfloat16)

    xc = (x_ref[0] - mu_ref[...]).astype(jnp.bfloat16)
    yb = _tmm(xc, w1_ref[...]).astype(jnp.bfloat16)
    p = jnp.maximum(yb, yb * alpha_ref[...])
    zb = jnp.dot(p, w2_ref[...],
                 preferred_element_type=jnp.float32).astype(jnp.bfloat16)
    zc_ref[...] = zb
    ones = jnp.ones((1, zb.shape[0]), jnp.bfloat16)
    s2_ref[...] += jnp.dot(ones, zb, preferred_element_type=jnp.float32)
    q2_ref[...] += jnp.dot(ones, zb * zb,
                           preferred_element_type=jnp.float32)


def _out_body(zc_ref, s2_ref, q2_ref, g2_ref, be2_ref, out_ref, *, inv_n):
    m = s2_ref[...] * inv_n
    v = q2_ref[...] * inv_n - m * m
    sc = g2_ref[...] * jax.lax.rsqrt(v + _EPS)
    sh = be2_ref[...] - m * sc
    out_ref[...] = zc_ref[...].astype(jnp.float32) * sc + sh


def kernel(last, extra, W1p, b1p, g1p, be1p, a1p, W2p, b2p, g2p, be2p,
           W1e, b1e, g1e, be1e, a1e, W2e, b2e, g2e, be2e):
    n = last.shape[0]
    blk = _BLK
    nb = n // blk
    assert nb * blk == n
    inv_n = 1.0 / n

    xp = last.reshape(nb, blk, 3).swapaxes(1, 2)
    ep = extra.reshape(nb, blk, 16).swapaxes(1, 2)
    x20 = jnp.concatenate(
        [xp, ep, jnp.ones((nb, 1, blk), jnp.float32)], axis=1)

    def fspec(shape):
        return pl.BlockSpec(shape, lambda i: (0, 0))

    xspec = pl.BlockSpec((1, 20, blk), lambda i: (i, 0, 0))

    params = dict(
        grid=(nb,),
        compiler_params=pltpu.CompilerParams(
            dimension_semantics=("arbitrary",)),
    )

    g20 = pl.pallas_call(
        _stats1_body,
        in_specs=[xspec],
        out_specs=fspec((20, 20)),
        out_shape=jax.ShapeDtypeStruct((20, 20), jnp.float32),
        **params,
    )(x20)

    be1_row = jnp.concatenate([be1p, be1e]).reshape(1, -1)
    alpha_row = jnp.concatenate(
        [jnp.full((1, 192), a1p, jnp.float32),
         jnp.full((1, 64), a1e, jnp.float32)],
        axis=1).astype(jnp.bfloat16)
    w2c = (jnp.pad(W2p.T, ((0, 64), (0, 32)))
           + jnp.pad(W2e.T, ((192, 0), (96, 0))))

    import functools as _ft
    s2, q2, zc = pl.pallas_call(
        _ft.partial(_stats2_body, inv_n=inv_n),
        in_specs=[xspec, fspec((20, 20)), fspec((3, 192)), fspec((16, 64)),
                  fspec((1, 192)), fspec((1, 64)), fspec((1, 256)),
                  fspec((1, 256)), fspec((256, 128))],
        out_specs=[fspec((1, 128)), fspec((1, 128)),
                   pl.BlockSpec((blk, 128), lambda i: (i, 0))],
        out_shape=[jax.ShapeDtypeStruct((1, 128), jnp.float32),
                   jax.ShapeDtypeStruct((1, 128), jnp.float32),
                   jax.ShapeDtypeStruct((n, 128), jnp.bfloat16)],
        scratch_shapes=[pltpu.VMEM((20, 1), jnp.float32),
                        pltpu.VMEM((20, 256), jnp.bfloat16)],
        **params,
    )(x20, g20, W1p.T, W1e.T, g1p.reshape(1, -1), g1e.reshape(1, -1),
      be1_row, alpha_row, w2c.astype(jnp.bfloat16))

    g2row = jnp.concatenate([g2p, g2e]).reshape(1, -1)
    be2row = jnp.concatenate([be2p, be2e]).reshape(1, -1)
    import functools as _ft
    return pl.pallas_call(
        _ft.partial(_out_body, inv_n=inv_n),
        in_specs=[pl.BlockSpec((blk, 128), lambda i: (i, 0)),
                  fspec((1, 128)), fspec((1, 128)),
                  fspec((1, 128)), fspec((1, 128))],
        out_specs=pl.BlockSpec((blk, 128), lambda i: (i, 0)),
        out_shape=jax.ShapeDtypeStruct((n, 128), jnp.float32),
        **params,
    )(zc, s2, q2, g2row, be2row)


# speedup vs baseline: 1.5460x; 1.0570x over previous
import jax
import jax.numpy as jnp
from jax.experimental import pallas as pl
from jax.experimental.pallas import tpu as pltpu

_EPS = 1e-5
_BLK = 25000


def _nt(a, b):
    return jax.lax.dot_general(a, b, (((1,), (1,)), ((), ())),
                               preferred_element_type=jnp.float32)


def _tmm(a, b):
    return jax.lax.dot_general(a, b, (((0,), (0,)), ((), ())),
                               preferred_element_type=jnp.float32)


def _stats1_body(x_ref, g_ref):
    i = pl.program_id(0)

    @pl.when(i == 0)
    def _():
        g_ref[...] = jnp.zeros_like(g_ref)

    xt = x_ref[0]
    g_ref[...] += _nt(xt, xt)


def _stats2_body(x_ref, g_ref, w1p_ref, w1e_ref, g1p_ref, g1e_ref,
                 be1_ref, alpha_ref, w2_ref, s2_ref, q2_ref, zc_ref,
                 mu_ref, w1_ref, *, inv_n):
    i = pl.program_id(0)

    @pl.when(i == 0)
    def _():
        s2_ref[...] = jnp.zeros_like(s2_ref)
        q2_ref[...] = jnp.zeros_like(q2_ref)
        g = g_ref[...] * inv_n
        mu_full = g[:, 19:20]
        keep = jax.lax.broadcasted_iota(jnp.int32, (20, 1), 0) < 19
        mu_ref[...] = jnp.where(keep, mu_full, 0.0)

        def scaled(cov, wT, gamma):
            var = jnp.sum(wT * (cov @ wT), axis=0, keepdims=True)
            a = gamma * jax.lax.rsqrt(var + _EPS)
            return (wT * a).astype(jnp.bfloat16)

        mux = mu_full[0:3]
        mue = mu_full[3:19]
        w1_ref[...] = jnp.zeros_like(w1_ref)
        w1_ref[0:3, 0:192] = scaled(g[0:3, 0:3] - mux @ mux.T,
                                    w1p_ref[...], g1p_ref[...])
        w1_ref[3:19, 192:256] = scaled(g[3:19, 3:19] - mue @ mue.T,
                                       w1e_ref[...], g1e_ref[...])
        w1_ref[19:20, :] = be1_ref[...].astype(jnp.bfloat16)

    xc = (x_ref[0] - mu_ref[...]).astype(jnp.bfloat16)
    yb = _tmm(xc, w1_ref[...]).astype(jnp.bfloat16)
    p = jnp.maximum(yb, yb * alpha_ref[...])
    zb = jnp.dot(p, w2_ref[...],
                 preferred_element_type=jnp.float32).astype(jnp.bfloat16)
    zc_ref[...] = zb
    ones = jnp.ones((1, zb.shape[0]), jnp.bfloat16)
    s2_ref[...] += jnp.dot(ones, zb, preferred_element_type=jnp.float32)
    q2_ref[...] += jnp.dot(ones, zb * zb,
                           preferred_element_type=jnp.float32)


def _out_body(zc_ref, s2_ref, q2_ref, g2_ref, be2_ref, out_ref, *, inv_n):
    m = s2_ref[...] * inv_n
    v = q2_ref[...] * inv_n - m * m
    sc = g2_ref[...] * jax.lax.rsqrt(v + _EPS)
    sh = be2_ref[...] - m * sc
    out_ref[...] = zc_ref[...].astype(jnp.float32) * sc + sh


def kernel(last, extra, W1p, b1p, g1p, be1p, a1p, W2p, b2p, g2p, be2p,
           W1e, b1e, g1e, be1e, a1e, W2e, b2e, g2e, be2e):
    n = last.shape[0]
    blk = _BLK
    nb = n // blk
    assert nb * blk == n
    inv_n = 1.0 / n

    xp = last.reshape(nb, blk, 3).swapaxes(1, 2)
    ep = extra.reshape(nb, blk, 16).swapaxes(1, 2)
    x20 = jnp.concatenate(
        [xp, ep, jnp.ones((nb, 1, blk), jnp.float32)], axis=1)

    def fspec(shape):
        return pl.BlockSpec(shape, lambda i: (0, 0))

    xspec = pl.BlockSpec((1, 20, blk), lambda i: (i, 0, 0))

    params = dict(
        grid=(nb,),
        compiler_params=pltpu.CompilerParams(
            dimension_semantics=("arbitrary",)),
    )

    g20 = pl.pallas_call(
        _stats1_body,
        in_specs=[xspec],
        out_specs=fspec((20, 20)),
        out_shape=jax.ShapeDtypeStruct((20, 20), jnp.float32),
        **params,
    )(x20)

    be1_row = jnp.concatenate([be1p, be1e]).reshape(1, -1)
    alpha_row = jnp.concatenate(
        [jnp.full((1, 192), a1p, jnp.float32),
         jnp.full((1, 64), a1e, jnp.float32)],
        axis=1).astype(jnp.bfloat16)
    w2c = (jnp.pad(W2p.T, ((0, 64), (0, 32)))
           + jnp.pad(W2e.T, ((192, 0), (96, 0))))

    import functools as _ft
    s2, q2, zc = pl.pallas_call(
        _ft.partial(_stats2_body, inv_n=inv_n),
        in_specs=[xspec, fspec((20, 20)), fspec((3, 192)), fspec((16, 64)),
                  fspec((1, 192)), fspec((1, 64)), fspec((1, 256)),
                  fspec((1, 256)), fspec((256, 128))],
        out_specs=[fspec((1, 128)), fspec((1, 128)),
                   pl.BlockSpec((blk, 128), lambda i: (i, 0))],
        out_shape=[jax.ShapeDtypeStruct((1, 128), jnp.float32),
                   jax.ShapeDtypeStruct((1, 128), jnp.float32),
                   jax.ShapeDtypeStruct((n, 128), jnp.bfloat16)],
        scratch_shapes=[pltpu.VMEM((20, 1), jnp.float32),
                        pltpu.VMEM((20, 256), jnp.bfloat16)],
        **params,
    )(x20, g20, W1p.T, W1e.T, g1p.reshape(1, -1), g1e.reshape(1, -1),
      be1_row, alpha_row, w2c.astype(jnp.bfloat16))

    g2row = jnp.concatenate([g2p, g2e]).reshape(1, -1)
    be2row = jnp.concatenate([be2p, be2e]).reshape(1, -1)
    import functools as _ft
    return pl.pallas_call(
        _ft.partial(_out_body, inv_n=inv_n),
        in_specs=[pl.BlockSpec((blk, 128), lambda i: (i, 0)),
                  fspec((1, 128)), fspec((1, 128)),
                  fspec((1, 128)), fspec((1, 128))],
        out_specs=pl.BlockSpec((blk, 128), lambda i: (i, 0)),
        out_shape=jax.ShapeDtypeStruct((n, 128), jnp.float32),
        **params,
    )(zc, s2, q2, g2row, be2row)
